# baseline (device time: 25801 ns/iter reference)
import jax
import jax.numpy as jnp
from jax import lax
from jax.experimental import pallas as pl
from jax.experimental.pallas import tpu as pltpu

N_DEV = 32
GROUP = 8
N_GROUPS = 4
N_SLOTS = GROUP + N_GROUPS
H = 2
Q = 4
C_GLOBAL = 16384
EPS = 1e-5


def kernel(x, t_emb, W_scale, W_shift):
    b, s, c = x.shape
    s_h = s // H
    s_q = s // Q

    def body(
        x_hbm, t_ref, ws_ref, wsh_ref, out_hbm,
        xbuf, obuf, comm_ref, in_sems, out_sems, send_sems, recv_sems,
    ):
        my_idx = lax.axis_index("i")
        my_g = my_idx // GROUP
        my_r = lax.rem(my_idx, GROUP)

        group_peers = [
            my_g * GROUP + lax.rem(my_r + o, GROUP) for o in range(1, GROUP)
        ]
        partners = [
            lax.rem(my_g + o2, N_GROUPS) * GROUP + my_r
            for o2 in range(1, N_GROUPS)
        ]

        barrier_sem = pltpu.get_barrier_semaphore()
        for tgt in group_peers + partners:
            pl.semaphore_signal(
                barrier_sem, inc=1,
                device_id=(tgt,), device_id_type=pl.DeviceIdType.MESH,
            )

        in_cp = []
        for q in range(Q):
            cp = pltpu.make_async_copy(
                x_hbm.at[:, q * s_q : (q + 1) * s_q, :], xbuf.at[q], in_sems.at[q]
            )
            cp.start()
            in_cp.append(cp)

        def stats_for(q):
            xq = xbuf[q]
            hh, off = divmod(q * s_q, s_h)
            comm_ref[hh, 0, 0:b, off : off + s_q] = jnp.sum(xq, axis=-1)
            comm_ref[hh, 0, b : 2 * b, off : off + s_q] = jnp.sum(xq * xq, axis=-1)

        def start_p1(hh):
            rds = []
            for o in range(1, GROUP):
                rdma = pltpu.make_async_remote_copy(
                    src_ref=comm_ref.at[hh, 0],
                    dst_ref=comm_ref.at[hh, o],
                    send_sem=send_sems.at[hh, o],
                    recv_sem=recv_sems.at[hh, o],
                    device_id=(group_peers[o - 1],),
                    device_id_type=pl.DeviceIdType.MESH,
                )
                rdma.start()
                rds.append(rdma)
            return rds

        def start_p2(hh):
            rds = []
            for o2 in range(1, N_GROUPS):
                rdma = pltpu.make_async_remote_copy(
                    src_ref=comm_ref.at[hh, GROUP],
                    dst_ref=comm_ref.at[hh, GROUP + o2],
                    send_sem=send_sems.at[hh, GROUP + o2],
                    recv_sem=recv_sems.at[hh, GROUP + o2],
                    device_id=(partners[o2 - 1],),
                    device_id_type=pl.DeviceIdType.MESH,
                )
                rdma.start()
                rds.append(rdma)
            return rds

        for q in range(Q):
            in_cp[q].wait()
            stats_for(q)
            if q == 1:
                pl.semaphore_wait(barrier_sem, (GROUP - 1) + (N_GROUPS - 1))
                p1_0 = start_p1(0)
        p1 = [p1_0, start_p1(1)]

        scale = jnp.dot(t_ref[...], ws_ref[...], preferred_element_type=jnp.float32)
        shift = jnp.dot(t_ref[...], wsh_ref[...], preferred_element_type=jnp.float32)

        p2 = [None] * H
        for hh in range(H):
            for r in p1[hh]:
                r.wait()
            comm_ref[hh, GROUP] = jnp.sum(comm_ref[hh, 0:GROUP], axis=0)
            p2[hh] = start_p2(hh)

        out_cp = [None] * Q
        for q in range(Q):
            hh, off = divmod(q * s_q, s_h)
            if off == 0:
                for r in p2[hh]:
                    r.wait()
                total = jnp.sum(comm_ref[hh, GROUP : GROUP + N_GROUPS], axis=0)
                mean_h = total[0:b] / C_GLOBAL
                var_h = total[b : 2 * b] / C_GLOBAL - mean_h * mean_h
                rstd_h = lax.rsqrt(var_h + EPS)
            mean = mean_h[:, off : off + s_q]
            rstd = rstd_h[:, off : off + s_q]
            if q >= 2:
                out_cp[q - 2].wait()
            hn = (xbuf[q] - mean[:, :, None]) * rstd[:, :, None]
            obuf[q % 2] = hn * (1.0 + scale[:, None, :]) + shift[:, None, :]
            cp = pltpu.make_async_copy(
                obuf.at[q % 2],
                out_hbm.at[:, q * s_q : (q + 1) * s_q, :],
                out_sems.at[q % 2],
            )
            cp.start()
            out_cp[q] = cp
        out_cp[Q - 2].wait()
        out_cp[Q - 1].wait()

    return pl.pallas_call(
        body,
        out_shape=jax.ShapeDtypeStruct((b, s, c), jnp.float32),
        in_specs=[
            pl.BlockSpec(memory_space=pltpu.MemorySpace.HBM),
            pl.BlockSpec(memory_space=pltpu.VMEM),
            pl.BlockSpec(memory_space=pltpu.VMEM),
            pl.BlockSpec(memory_space=pltpu.VMEM),
        ],
        out_specs=pl.BlockSpec(memory_space=pltpu.MemorySpace.HBM),
        scratch_shapes=[
            pltpu.VMEM((Q, b, s_q, c), jnp.float32),
            pltpu.VMEM((2, b, s_q, c), jnp.float32),
            pltpu.VMEM((H, N_SLOTS, 2 * b, s_h), jnp.float32),
            pltpu.SemaphoreType.DMA((Q,)),
            pltpu.SemaphoreType.DMA((2,)),
            pltpu.SemaphoreType.DMA((H, N_SLOTS)),
            pltpu.SemaphoreType.DMA((H, N_SLOTS)),
        ],
        compiler_params=pltpu.CompilerParams(collective_id=0),
    )(x, t_emb, W_scale, W_shift)


# device time: 24647 ns/iter; 1.0468x vs baseline; 1.0468x over previous
import jax
import jax.numpy as jnp
from jax import lax
from jax.experimental import pallas as pl
from jax.experimental.pallas import tpu as pltpu

N_DEV = 32
GROUP = 8
N_GROUPS = 4
N_SLOTS = GROUP + N_GROUPS
H = 2
C_GLOBAL = 16384
EPS = 1e-5


def kernel(x, t_emb, W_scale, W_shift):
    b, s, c = x.shape
    s_h = s // H

    def body(
        x_ref, t_ref, ws_ref, wsh_ref, out_hbm,
        obuf, comm_ref, out_sems, send_sems, recv_sems,
    ):
        my_idx = lax.axis_index("i")
        my_g = my_idx // GROUP
        my_r = lax.rem(my_idx, GROUP)

        group_peers = [
            my_g * GROUP + lax.rem(my_r + o, GROUP) for o in range(1, GROUP)
        ]
        partners = [
            lax.rem(my_g + o2, N_GROUPS) * GROUP + my_r
            for o2 in range(1, N_GROUPS)
        ]

        barrier_sem = pltpu.get_barrier_semaphore()
        for tgt in group_peers + partners:
            pl.semaphore_signal(
                barrier_sem, inc=1,
                device_id=(tgt,), device_id_type=pl.DeviceIdType.MESH,
            )

        def stats_for(hh):
            xh = x_ref[:, hh * s_h : (hh + 1) * s_h, :]
            s1 = jnp.sum(xh, axis=-1)
            s2 = jnp.sum(xh * xh, axis=-1)
            comm_ref[hh, 0] = jnp.concatenate([s1, s2], axis=0)

        def start_p1(hh):
            rds = []
            for o in range(1, GROUP):
                rdma = pltpu.make_async_remote_copy(
                    src_ref=comm_ref.at[hh, 0],
                    dst_ref=comm_ref.at[hh, o],
                    send_sem=send_sems.at[hh, o],
                    recv_sem=recv_sems.at[hh, o],
                    device_id=(group_peers[o - 1],),
                    device_id_type=pl.DeviceIdType.MESH,
                )
                rdma.start()
                rds.append(rdma)
            return rds

        def start_p2(hh):
            rds = []
            for o2 in range(1, N_GROUPS):
                rdma = pltpu.make_async_remote_copy(
                    src_ref=comm_ref.at[hh, GROUP],
                    dst_ref=comm_ref.at[hh, GROUP + o2],
                    send_sem=send_sems.at[hh, GROUP + o2],
                    recv_sem=recv_sems.at[hh, GROUP + o2],
                    device_id=(partners[o2 - 1],),
                    device_id_type=pl.DeviceIdType.MESH,
                )
                rdma.start()
                rds.append(rdma)
            return rds

        def apply(hh, scale, shift):
            total = jnp.sum(comm_ref[hh, GROUP : GROUP + N_GROUPS], axis=0)
            mean = total[0:b] / C_GLOBAL
            var = total[b : 2 * b] / C_GLOBAL - mean * mean
            rstd = lax.rsqrt(var + EPS)
            xh = x_ref[:, hh * s_h : (hh + 1) * s_h, :]
            hn = (xh - mean[:, :, None]) * rstd[:, :, None]
            obuf[hh] = hn * (1.0 + scale[:, None, :]) + shift[:, None, :]
            cp = pltpu.make_async_copy(
                obuf.at[hh],
                out_hbm.at[:, hh * s_h : (hh + 1) * s_h, :],
                out_sems.at[hh],
            )
            cp.start()
            return cp

        stats_for(0)
        pl.semaphore_wait(barrier_sem, (GROUP - 1) + (N_GROUPS - 1))
        p1 = [None] * H
        p2 = [None] * H
        p1[0] = start_p1(0)

        stats_for(1)
        p1[1] = start_p1(1)

        scale = jnp.dot(t_ref[...], ws_ref[...], preferred_element_type=jnp.float32)
        shift = jnp.dot(t_ref[...], wsh_ref[...], preferred_element_type=jnp.float32)

        for hh in range(H):
            for r in p1[hh]:
                r.wait()
            comm_ref[hh, GROUP] = jnp.sum(comm_ref[hh, 0:GROUP], axis=0)
            p2[hh] = start_p2(hh)

        out_cp = [None] * H
        for hh in range(H):
            for r in p2[hh]:
                r.wait()
            out_cp[hh] = apply(hh, scale, shift)
        for cp in out_cp:
            cp.wait()

    return pl.pallas_call(
        body,
        out_shape=jax.ShapeDtypeStruct((b, s, c), jnp.float32),
        in_specs=[
            pl.BlockSpec(memory_space=pltpu.VMEM),
            pl.BlockSpec(memory_space=pltpu.VMEM),
            pl.BlockSpec(memory_space=pltpu.VMEM),
            pl.BlockSpec(memory_space=pltpu.VMEM),
        ],
        out_specs=pl.BlockSpec(memory_space=pltpu.MemorySpace.HBM),
        scratch_shapes=[
            pltpu.VMEM((H, b, s_h, c), jnp.float32),
            pltpu.VMEM((H, N_SLOTS, 2 * b, s_h), jnp.float32),
            pltpu.SemaphoreType.DMA((H,)),
            pltpu.SemaphoreType.DMA((H, N_SLOTS)),
            pltpu.SemaphoreType.DMA((H, N_SLOTS)),
        ],
        compiler_params=pltpu.CompilerParams(collective_id=0),
    )(x, t_emb, W_scale, W_shift)


# device time: 24298 ns/iter; 1.0619x vs baseline; 1.0144x over previous
import jax
import jax.numpy as jnp
from jax import lax
from jax.experimental import pallas as pl
from jax.experimental.pallas import tpu as pltpu

N_DEV = 32
GROUP = 8
N_GROUPS = 4
N_SLOTS = GROUP + N_GROUPS
H = 2
C_GLOBAL = 16384
EPS = 1e-5


def kernel(x, t_emb, W_scale, W_shift):
    b, s, c = x.shape
    s_h = s // H

    def body(x_ref, t_ref, ws_ref, wsh_ref, out_ref, comm_ref, send_sems, recv_sems):
        my_idx = lax.axis_index("i")
        my_g = my_idx // GROUP
        my_r = lax.rem(my_idx, GROUP)

        group_peers = [
            my_g * GROUP + lax.rem(my_r + o, GROUP) for o in range(1, GROUP)
        ]
        partners = [
            lax.rem(my_g + o2, N_GROUPS) * GROUP + my_r
            for o2 in range(1, N_GROUPS)
        ]

        barrier_sem = pltpu.get_barrier_semaphore()
        for tgt in group_peers + partners:
            pl.semaphore_signal(
                barrier_sem, inc=1,
                device_id=(tgt,), device_id_type=pl.DeviceIdType.MESH,
            )

        def stats_for(hh):
            xh = x_ref[:, hh * s_h : (hh + 1) * s_h, :]
            s1 = jnp.sum(xh, axis=-1)
            s2 = jnp.sum(xh * xh, axis=-1)
            comm_ref[hh, 0] = jnp.concatenate([s1, s2], axis=0)

        def start_p1(hh):
            rds = []
            for o in range(1, GROUP):
                rdma = pltpu.make_async_remote_copy(
                    src_ref=comm_ref.at[hh, 0],
                    dst_ref=comm_ref.at[hh, o],
                    send_sem=send_sems.at[hh, o],
                    recv_sem=recv_sems.at[hh, o],
                    device_id=(group_peers[o - 1],),
                    device_id_type=pl.DeviceIdType.MESH,
                )
                rdma.start()
                rds.append(rdma)
            return rds

        def start_p2(hh):
            rds = []
            for o2 in range(1, N_GROUPS):
                rdma = pltpu.make_async_remote_copy(
                    src_ref=comm_ref.at[hh, GROUP],
                    dst_ref=comm_ref.at[hh, GROUP + o2],
                    send_sem=send_sems.at[hh, GROUP + o2],
                    recv_sem=recv_sems.at[hh, GROUP + o2],
                    device_id=(partners[o2 - 1],),
                    device_id_type=pl.DeviceIdType.MESH,
                )
                rdma.start()
                rds.append(rdma)
            return rds

        def apply(hh, scale, shift):
            total = jnp.sum(comm_ref[hh, GROUP : GROUP + N_GROUPS], axis=0)
            mean = total[0:b] / C_GLOBAL
            var = total[b : 2 * b] / C_GLOBAL - mean * mean
            rstd = lax.rsqrt(var + EPS)
            xh = x_ref[:, hh * s_h : (hh + 1) * s_h, :]
            hn = (xh - mean[:, :, None]) * rstd[:, :, None]
            out_ref[:, hh * s_h : (hh + 1) * s_h, :] = (
                hn * (1.0 + scale[:, None, :]) + shift[:, None, :]
            )

        stats_for(0)
        pl.semaphore_wait(barrier_sem, (GROUP - 1) + (N_GROUPS - 1))
        p1 = [None] * H
        p2 = [None] * H
        p1[0] = start_p1(0)

        stats_for(1)
        p1[1] = start_p1(1)

        scale = jnp.dot(t_ref[...], ws_ref[...], preferred_element_type=jnp.float32)
        shift = jnp.dot(t_ref[...], wsh_ref[...], preferred_element_type=jnp.float32)

        for hh in range(H):
            for r in p1[hh]:
                r.wait()
            comm_ref[hh, GROUP] = jnp.sum(comm_ref[hh, 0:GROUP], axis=0)
            p2[hh] = start_p2(hh)

        for hh in range(H):
            for r in p2[hh]:
                r.wait()
            apply(hh, scale, shift)

    return pl.pallas_call(
        body,
        out_shape=jax.ShapeDtypeStruct((b, s, c), jnp.float32),
        in_specs=[pl.BlockSpec(memory_space=pltpu.VMEM)] * 4,
        out_specs=pl.BlockSpec(memory_space=pltpu.VMEM),
        scratch_shapes=[
            pltpu.VMEM((H, N_SLOTS, 2 * b, s_h), jnp.float32),
            pltpu.SemaphoreType.DMA((H, N_SLOTS)),
            pltpu.SemaphoreType.DMA((H, N_SLOTS)),
        ],
        compiler_params=pltpu.CompilerParams(collective_id=0),
    )(x, t_emb, W_scale, W_shift)
